# Initial kernel scaffold; baseline (speedup 1.0000x reference)
import jax
import jax.numpy as jnp
from jax.experimental import pallas as pl


def kernel(feat, edge_index, W, b):
    raise NotImplementedError("write your pallas kernel here")


# speedup vs baseline: 4.0316x; 4.0316x over previous
import jax
import jax.numpy as jnp
from jax import lax
from jax.experimental import pallas as pl
from jax.experimental.pallas import tpu as pltpu
from jax.experimental.pallas import tpu_sc as plsc

NC = 2
NS = 16
NW = NC * NS
CHUNK = 80
NP = 10240


def _sc_mesh():
    return plsc.VectorSubcoreMesh(core_axis_name="c", subcore_axis_name="s")


def _deg_call(dst, zeros1d, E):
    n_it = E // (NW * CHUNK)
    per_w = E // NW
    rps = NP // NS

    def body(dst_hbm, zeros_hbm, out0_hbm, out1_hbm,
             ones_v, didx_v, acc):
        c = lax.axis_index("c")
        s = lax.axis_index("s")
        w = s * NC + c
        sl = pl.ds(s * rps, rps)
        pltpu.sync_copy(zeros_hbm.at[sl], acc.at[sl])
        for i in range(CHUNK // 16):
            ones_v[pl.ds(i * 16, 16)] = jnp.ones((16,), jnp.float32)
        plsc.subcore_barrier()

        def it(i, carry):
            base = w * per_w + i * CHUNK
            pltpu.sync_copy(dst_hbm.at[pl.ds(base, CHUNK)], didx_v)
            pltpu.sync_copy(ones_v, acc.at[didx_v], add=True)
            return carry

        lax.fori_loop(0, n_it, it, 0)
        plsc.subcore_barrier()

        @pl.when(c == 0)
        def _():
            pltpu.sync_copy(acc.at[sl], out0_hbm.at[sl])

        @pl.when(c == 1)
        def _():
            pltpu.sync_copy(acc.at[sl], out1_hbm.at[sl])

    f = pl.kernel(
        body,
        out_type=(
            jax.ShapeDtypeStruct((NP,), jnp.float32),
            jax.ShapeDtypeStruct((NP,), jnp.float32),
        ),
        mesh=_sc_mesh(),
        scratch_types=[
            pltpu.VMEM((CHUNK,), jnp.float32),
            pltpu.VMEM((CHUNK,), jnp.int32),
            pltpu.VMEM_SHARED((NP,), jnp.float32),
        ],
    )
    return f(dst, zeros1d)


def _hop_call(hs, src, dst, zeros128, E):
    n_it = E // (NW * CHUNK)
    per_w = E // NW
    rps = NP // NS

    def body(hs_hbm, src_hbm, dst_hbm, zeros_hbm, out0_hbm, out1_hbm,
             sidx_v, didx_v, rows_v, acc, sem):
        c = lax.axis_index("c")
        s = lax.axis_index("s")
        w = s * NC + c
        sl = pl.ds(s * rps, rps)
        pltpu.sync_copy(zeros_hbm.at[sl], acc.at[sl])
        plsc.subcore_barrier()

        def it(i, carry):
            base = w * per_w + i * CHUNK
            pltpu.sync_copy(src_hbm.at[pl.ds(base, CHUNK)], sidx_v)
            pltpu.sync_copy(dst_hbm.at[pl.ds(base, CHUNK)], didx_v)
            pltpu.async_copy(hs_hbm.at[sidx_v], rows_v, sem).wait()
            pltpu.sync_copy(rows_v, acc.at[didx_v], add=True)
            return carry

        lax.fori_loop(0, n_it, it, 0)
        plsc.subcore_barrier()

        @pl.when(c == 0)
        def _():
            pltpu.sync_copy(acc.at[sl], out0_hbm.at[sl])

        @pl.when(c == 1)
        def _():
            pltpu.sync_copy(acc.at[sl], out1_hbm.at[sl])

    f = pl.kernel(
        body,
        out_type=(
            jax.ShapeDtypeStruct((NP, 128), jnp.float32),
            jax.ShapeDtypeStruct((NP, 128), jnp.float32),
        ),
        mesh=_sc_mesh(),
        scratch_types=[
            pltpu.VMEM((CHUNK,), jnp.int32),
            pltpu.VMEM((CHUNK,), jnp.int32),
            pltpu.VMEM((CHUNK, 128), jnp.float32),
            pltpu.VMEM_SHARED((NP, 128), jnp.float32),
            pltpu.SemaphoreType.DMA,
        ],
    )
    return f(hs, src, dst, zeros128)


def _norm_scale_call(deg0, deg1, feat, N):
    R = 400
    grid = (N // R,)

    def body(d0_ref, d1_ref, feat_ref, norm_ref, hs_ref):
        d = d0_ref[...] + d1_ref[...]
        nrm = jnp.where(d > 0.0, lax.rsqrt(d), 0.0)
        norm_ref[...] = nrm
        hs_ref[...] = feat_ref[...] * nrm

    return pl.pallas_call(
        body,
        grid=grid,
        in_specs=[
            pl.BlockSpec((R, 1), lambda i: (i, 0)),
            pl.BlockSpec((R, 1), lambda i: (i, 0)),
            pl.BlockSpec((R, 128), lambda i: (i, 0)),
        ],
        out_specs=[
            pl.BlockSpec((R, 1), lambda i: (i, 0)),
            pl.BlockSpec((R, 128), lambda i: (i, 0)),
        ],
        out_shape=[
            jax.ShapeDtypeStruct((N, 1), jnp.float32),
            jax.ShapeDtypeStruct((N, 128), jnp.float32),
        ],
    )(deg0, deg1, feat)


def _combine_scale_call(a0, a1, norm, N):
    R = 400
    grid = (N // R,)

    def body(a0_ref, a1_ref, norm_ref, hs_ref):
        nrm = norm_ref[...]
        hs_ref[...] = (a0_ref[...] + a1_ref[...]) * (nrm * nrm)

    return pl.pallas_call(
        body,
        grid=grid,
        in_specs=[
            pl.BlockSpec((R, 128), lambda i: (i, 0)),
            pl.BlockSpec((R, 128), lambda i: (i, 0)),
            pl.BlockSpec((R, 1), lambda i: (i, 0)),
        ],
        out_specs=pl.BlockSpec((R, 128), lambda i: (i, 0)),
        out_shape=jax.ShapeDtypeStruct((N, 128), jnp.float32),
    )(a0, a1, norm)


def _final_call(b0, b1, norm, W, b2, N):
    R = 400
    grid = (N // R,)

    def body(b0_ref, b1_ref, norm_ref, w_ref, bias_ref, out_ref):
        h = (b0_ref[...] + b1_ref[...]) * norm_ref[...]
        out_ref[...] = lax.dot_general(
            h, w_ref[...], (((1,), (1,)), ((), ())),
            preferred_element_type=jnp.float32) + bias_ref[...]

    return pl.pallas_call(
        body,
        grid=grid,
        in_specs=[
            pl.BlockSpec((R, 128), lambda i: (i, 0)),
            pl.BlockSpec((R, 128), lambda i: (i, 0)),
            pl.BlockSpec((R, 1), lambda i: (i, 0)),
            pl.BlockSpec((128, 128), lambda i: (0, 0)),
            pl.BlockSpec((1, 128), lambda i: (0, 0)),
        ],
        out_specs=pl.BlockSpec((R, 128), lambda i: (i, 0)),
        out_shape=jax.ShapeDtypeStruct((N, 128), jnp.float32),
    )(b0, b1, norm, W, b2)


def kernel(feat, edge_index, W, b):
    N, D = feat.shape
    E = edge_index.shape[1]
    ei = edge_index.astype(jnp.int32)
    src, dst = ei[0], ei[1]
    zeros128 = jnp.zeros((NP, 128), jnp.float32)
    zeros1d = jnp.zeros((NP,), jnp.float32)

    deg0, deg1 = _deg_call(dst, zeros1d, E)
    norm, hs1 = _norm_scale_call(deg0.reshape(NP, 1), deg1.reshape(NP, 1),
                                 feat, N)
    a0, a1 = _hop_call(hs1, src, dst, zeros128, E)
    hs2 = _combine_scale_call(a0, a1, norm, N)
    b0, b1 = _hop_call(hs2, src, dst, zeros128, E)
    return _final_call(b0, b1, norm, W, b.reshape(1, 128), N)


# speedup vs baseline: 9.7738x; 2.4243x over previous
import jax
import jax.numpy as jnp
from jax import lax
from jax.experimental import pallas as pl
from jax.experimental.pallas import tpu as pltpu
from jax.experimental.pallas import tpu_sc as plsc

NC = 2
NS = 16
NW = NC * NS
CHUNK = 128
NP = 10240


def _sc_mesh():
    return plsc.VectorSubcoreMesh(core_axis_name="c", subcore_axis_name="s")


def _pad_edges(src, dst, E, N, n_it):
    e_pad = NW * n_it * CHUNK
    npad = e_pad - E
    fill = jnp.arange(npad, dtype=jnp.int32)
    srcp = jnp.concatenate([src, fill % N])
    dstp = jnp.concatenate([dst, N + (fill % (NP - N))])
    return (srcp.reshape(e_pad // CHUNK, CHUNK),
            dstp.reshape(e_pad // CHUNK, CHUNK), dstp)


def _deg_call(dst2d, zeros1d):
    n_it = dst2d.shape[0] // NW
    rps = NP // NS

    def body(dst_hbm, zeros_hbm, out0_hbm, out1_hbm,
             ones_v, didx2, acc, sem):
        c = lax.axis_index("c")
        s = lax.axis_index("s")
        w = s * NC + c
        sl = pl.ds(s * rps, rps)
        pltpu.sync_copy(zeros_hbm.at[sl], acc.at[sl])
        for i in range(CHUNK // 16):
            ones_v[pl.ds(i * 16, 16)] = jnp.ones((16,), jnp.float32)
        pltpu.sync_copy(dst_hbm.at[pl.ds(w * n_it, n_it)], didx2)
        plsc.subcore_barrier()

        grp = 8

        def it(k, carry):
            for j in range(grp):
                pltpu.async_copy(ones_v, acc.at[didx2.at[k * grp + j]],
                                 sem, add=True)
            for j in range(grp):
                pltpu.make_async_copy(ones_v, acc.at[didx2.at[k * grp + j]],
                                      sem).wait()
            return carry

        lax.fori_loop(0, n_it // grp, it, 0)
        plsc.subcore_barrier()

        @pl.when(c == 0)
        def _():
            pltpu.sync_copy(acc.at[sl], out0_hbm.at[sl])

        @pl.when(c == 1)
        def _():
            pltpu.sync_copy(acc.at[sl], out1_hbm.at[sl])

    f = pl.kernel(
        body,
        out_type=(
            jax.ShapeDtypeStruct((NP,), jnp.float32),
            jax.ShapeDtypeStruct((NP,), jnp.float32),
        ),
        mesh=_sc_mesh(),
        scratch_types=[
            pltpu.VMEM((CHUNK,), jnp.float32),
            pltpu.VMEM((n_it, CHUNK), jnp.int32),
            pltpu.VMEM_SHARED((NP,), jnp.float32),
            pltpu.SemaphoreType.DMA,
        ],
    )
    return f(dst2d, zeros1d)


def _hop_call(hs, src2d, dst1d, zeros128):
    n_it = src2d.shape[0] // NW
    rps = NP // NS

    def body(hs_hbm, src_hbm, dst_hbm, zeros_hbm, out0_hbm, out1_hbm,
             sidx2, didx0, didx1, rows0, rows1, acc,
             sem0, sem1, dsem0, dsem1):
        c = lax.axis_index("c")
        s = lax.axis_index("s")
        w = s * NC + c
        sl = pl.ds(s * rps, rps)
        pltpu.sync_copy(zeros_hbm.at[sl], acc.at[sl])
        pltpu.sync_copy(src_hbm.at[pl.ds(w * n_it, n_it)], sidx2)
        plsc.subcore_barrier()

        rows = (rows0, rows1)
        sems = (sem0, sem1)
        didx = (didx0, didx1)
        dsems = (dsem0, dsem1)
        ebase = w * n_it * CHUNK

        def dslice(i):
            return dst_hbm.at[pl.ds(ebase + i * CHUNK, CHUNK)]

        pltpu.async_copy(hs_hbm.at[sidx2.at[0]], rows0, sem0)
        pltpu.async_copy(dslice(0), didx0, dsem0)

        def pair(k, carry):
            for b in range(2):
                i = k * 2 + b
                nb = 1 - b
                ip1 = lax.rem(i + 1, n_it)
                pltpu.async_copy(hs_hbm.at[sidx2.at[ip1]], rows[nb],
                                 sems[nb])
                pltpu.async_copy(dslice(ip1), didx[nb], dsems[nb])
                pltpu.make_async_copy(dslice(i), didx[b], dsems[b]).wait()
                pltpu.make_async_copy(hs_hbm.at[sidx2.at[i]], rows[b],
                                      sems[b]).wait()
                pltpu.sync_copy(rows[b], acc.at[didx[b]], add=True)
            return carry

        lax.fori_loop(0, n_it // 2, pair, 0)
        pltpu.make_async_copy(dslice(0), didx[0], dsems[0]).wait()
        pltpu.make_async_copy(hs_hbm.at[sidx2.at[0]], rows[0],
                              sems[0]).wait()
        plsc.subcore_barrier()

        @pl.when(c == 0)
        def _():
            pltpu.sync_copy(acc.at[sl], out0_hbm.at[sl])

        @pl.when(c == 1)
        def _():
            pltpu.sync_copy(acc.at[sl], out1_hbm.at[sl])

    f = pl.kernel(
        body,
        out_type=(
            jax.ShapeDtypeStruct((NP, 128), jnp.float32),
            jax.ShapeDtypeStruct((NP, 128), jnp.float32),
        ),
        mesh=_sc_mesh(),
        scratch_types=[
            pltpu.VMEM((n_it, CHUNK), jnp.int32),
            pltpu.VMEM((CHUNK,), jnp.int32),
            pltpu.VMEM((CHUNK,), jnp.int32),
            pltpu.VMEM((CHUNK, 128), jnp.float32),
            pltpu.VMEM((CHUNK, 128), jnp.float32),
            pltpu.VMEM_SHARED((NP, 128), jnp.float32),
            pltpu.SemaphoreType.DMA,
            pltpu.SemaphoreType.DMA,
            pltpu.SemaphoreType.DMA,
            pltpu.SemaphoreType.DMA,
        ],
    )
    return f(hs, src2d, dst1d, zeros128)


def _norm_scale_call(deg0, deg1, feat, N):
    R = 400
    grid = (N // R,)

    def body(d0_ref, d1_ref, feat_ref, norm_ref, hs_ref):
        d = d0_ref[...] + d1_ref[...]
        nrm = jnp.where(d > 0.0, lax.rsqrt(d), 0.0)
        norm_ref[...] = nrm
        hs_ref[...] = feat_ref[...] * nrm

    return pl.pallas_call(
        body,
        grid=grid,
        in_specs=[
            pl.BlockSpec((R, 1), lambda i: (i, 0)),
            pl.BlockSpec((R, 1), lambda i: (i, 0)),
            pl.BlockSpec((R, 128), lambda i: (i, 0)),
        ],
        out_specs=[
            pl.BlockSpec((R, 1), lambda i: (i, 0)),
            pl.BlockSpec((R, 128), lambda i: (i, 0)),
        ],
        out_shape=[
            jax.ShapeDtypeStruct((N, 1), jnp.float32),
            jax.ShapeDtypeStruct((N, 128), jnp.float32),
        ],
    )(deg0, deg1, feat)


def _combine_scale_call(a0, a1, norm, N):
    R = 400
    grid = (N // R,)

    def body(a0_ref, a1_ref, norm_ref, hs_ref):
        nrm = norm_ref[...]
        hs_ref[...] = (a0_ref[...] + a1_ref[...]) * (nrm * nrm)

    return pl.pallas_call(
        body,
        grid=grid,
        in_specs=[
            pl.BlockSpec((R, 128), lambda i: (i, 0)),
            pl.BlockSpec((R, 128), lambda i: (i, 0)),
            pl.BlockSpec((R, 1), lambda i: (i, 0)),
        ],
        out_specs=pl.BlockSpec((R, 128), lambda i: (i, 0)),
        out_shape=jax.ShapeDtypeStruct((N, 128), jnp.float32),
    )(a0, a1, norm)


def _final_call(b0, b1, norm, W, b2, N):
    R = 400
    grid = (N // R,)

    def body(b0_ref, b1_ref, norm_ref, w_ref, bias_ref, out_ref):
        h = (b0_ref[...] + b1_ref[...]) * norm_ref[...]
        out_ref[...] = lax.dot_general(
            h, w_ref[...], (((1,), (1,)), ((), ())),
            preferred_element_type=jnp.float32) + bias_ref[...]

    return pl.pallas_call(
        body,
        grid=grid,
        in_specs=[
            pl.BlockSpec((R, 128), lambda i: (i, 0)),
            pl.BlockSpec((R, 128), lambda i: (i, 0)),
            pl.BlockSpec((R, 1), lambda i: (i, 0)),
            pl.BlockSpec((128, 128), lambda i: (0, 0)),
            pl.BlockSpec((1, 128), lambda i: (0, 0)),
        ],
        out_specs=pl.BlockSpec((R, 128), lambda i: (i, 0)),
        out_shape=jax.ShapeDtypeStruct((N, 128), jnp.float32),
    )(b0, b1, norm, W, b2)


def kernel(feat, edge_index, W, b):
    N, D = feat.shape
    E = edge_index.shape[1]
    ei = edge_index.astype(jnp.int32)
    src, dst = ei[0], ei[1]
    zeros128 = jnp.zeros((NP, 128), jnp.float32)
    zeros1d = jnp.zeros((NP,), jnp.float32)

    n_it = -(-E // (NW * CHUNK * 8)) * 8
    src2d, dst2d, dst1d = _pad_edges(src, dst, E, N, n_it)

    deg0, deg1 = _deg_call(dst2d, zeros1d)
    norm, hs1 = _norm_scale_call(deg0.reshape(NP, 1), deg1.reshape(NP, 1),
                                 feat, N)
    a0, a1 = _hop_call(hs1, src2d, dst1d, zeros128)
    hs2 = _combine_scale_call(a0, a1, norm, N)
    b0, b1 = _hop_call(hs2, src2d, dst1d, zeros128)
    return _final_call(b0, b1, norm, W, b.reshape(1, 128), N)


# speedup vs baseline: 10.0896x; 1.0323x over previous
import jax
import jax.numpy as jnp
from jax import lax
from jax.experimental import pallas as pl
from jax.experimental.pallas import tpu as pltpu
from jax.experimental.pallas import tpu_sc as plsc

NC = 2
NS = 16
NW = NC * NS
CHUNK = 128
NP = 10240


def _sc_mesh():
    return plsc.VectorSubcoreMesh(core_axis_name="c", subcore_axis_name="s")


def _pad_dst(dst, E, N, n_it):
    e_pad = NW * n_it * CHUNK
    fill = jnp.arange(e_pad - E, dtype=jnp.int32)
    dstp = jnp.concatenate([dst, N + (fill % (NP - N))])
    return dstp.reshape(e_pad // CHUNK, CHUNK)


def _deg_call(dst2d, zeros1d):
    n_it = dst2d.shape[0] // NW
    rps = NP // NS

    def body(dst_hbm, zeros_hbm, out0_hbm, out1_hbm,
             ones_v, didx2, acc, sem):
        c = lax.axis_index("c")
        s = lax.axis_index("s")
        w = s * NC + c
        sl = pl.ds(s * rps, rps)
        pltpu.sync_copy(zeros_hbm.at[sl], acc.at[sl])
        for i in range(CHUNK // 16):
            ones_v[pl.ds(i * 16, 16)] = jnp.ones((16,), jnp.float32)
        pltpu.sync_copy(dst_hbm.at[pl.ds(w * n_it, n_it)], didx2)
        plsc.subcore_barrier()

        grp = 8

        def it(k, carry):
            for j in range(grp):
                pltpu.async_copy(ones_v, acc.at[didx2.at[k * grp + j]],
                                 sem, add=True)
            for j in range(grp):
                pltpu.make_async_copy(ones_v, acc.at[didx2.at[k * grp + j]],
                                      sem).wait()
            return carry

        lax.fori_loop(0, n_it // grp, it, 0)
        plsc.subcore_barrier()

        @pl.when(c == 0)
        def _():
            pltpu.sync_copy(acc.at[sl], out0_hbm.at[sl])

        @pl.when(c == 1)
        def _():
            pltpu.sync_copy(acc.at[sl], out1_hbm.at[sl])

    f = pl.kernel(
        body,
        out_type=(
            jax.ShapeDtypeStruct((NP,), jnp.float32),
            jax.ShapeDtypeStruct((NP,), jnp.float32),
        ),
        mesh=_sc_mesh(),
        scratch_types=[
            pltpu.VMEM((CHUNK,), jnp.float32),
            pltpu.VMEM((n_it, CHUNK), jnp.int32),
            pltpu.VMEM_SHARED((NP,), jnp.float32),
            pltpu.SemaphoreType.DMA,
        ],
    )
    return f(dst2d, zeros1d)


def _hop_call(hs, src1d, dst1d, E):
    per_w = E // NW
    n_full = per_w // CHUNK
    tail = per_w % CHUNK
    rps = NP // NS

    def body(hs_hbm, src_hbm, dst_hbm, out0_hbm, out1_hbm,
             sidx_buf, didx0, didx1, didx_t, rows0, rows1, acc,
             sem0, sem1, dsem0, dsem1, tsem):
        c = lax.axis_index("c")
        s = lax.axis_index("s")
        w = s * NC + c
        sl = pl.ds(s * rps, rps)
        ebase = w * per_w

        def zrow(r, carry):
            for j in range(8):
                rows0[r, pl.ds(j * 16, 16)] = jnp.zeros((16,), jnp.float32)
            return carry

        lax.fori_loop(0, CHUNK, zrow, 0)
        for k in range(rps // CHUNK):
            pltpu.sync_copy(rows0, acc.at[pl.ds(s * rps + k * CHUNK, CHUNK)])
        pltpu.sync_copy(src_hbm.at[pl.ds(ebase, per_w)], sidx_buf)
        plsc.subcore_barrier()

        rows = (rows0, rows1)
        sems = (sem0, sem1)
        didx = (didx0, didx1)
        dsems = (dsem0, dsem1)

        def sslice(i):
            return sidx_buf.at[pl.ds(i * CHUNK, CHUNK)]

        def dslice(i):
            return dst_hbm.at[pl.ds(ebase + i * CHUNK, CHUNK)]

        pltpu.async_copy(hs_hbm.at[sslice(0)], rows0, sem0)
        pltpu.async_copy(dslice(0), didx0, dsem0)

        def pair(k, carry):
            for b in range(2):
                i = k * 2 + b
                nb = 1 - b
                ip1 = lax.rem(i + 1, n_full)
                pltpu.async_copy(hs_hbm.at[sslice(ip1)], rows[nb],
                                 sems[nb])
                pltpu.async_copy(dslice(ip1), didx[nb], dsems[nb])
                pltpu.make_async_copy(dslice(i), didx[b], dsems[b]).wait()
                pltpu.make_async_copy(hs_hbm.at[sslice(i)], rows[b],
                                      sems[b]).wait()
                pltpu.sync_copy(rows[b], acc.at[didx[b]], add=True)
            return carry

        lax.fori_loop(0, n_full // 2, pair, 0)
        pltpu.make_async_copy(dslice(0), didx[0], dsems[0]).wait()
        pltpu.make_async_copy(hs_hbm.at[sslice(0)], rows[0],
                              sems[0]).wait()
        if tail:
            tbase = ebase + n_full * CHUNK
            pltpu.sync_copy(dst_hbm.at[pl.ds(tbase, tail)], didx_t)
            pltpu.async_copy(
                hs_hbm.at[sidx_buf.at[pl.ds(n_full * CHUNK, tail)]],
                rows1.at[pl.ds(0, tail)], tsem).wait()
            pltpu.sync_copy(rows1.at[pl.ds(0, tail)], acc.at[didx_t],
                            add=True)
        plsc.subcore_barrier()

        @pl.when(c == 0)
        def _():
            pltpu.sync_copy(acc.at[sl], out0_hbm.at[sl])

        @pl.when(c == 1)
        def _():
            pltpu.sync_copy(acc.at[sl], out1_hbm.at[sl])

    f = pl.kernel(
        body,
        out_type=(
            jax.ShapeDtypeStruct((NP, 128), jnp.float32),
            jax.ShapeDtypeStruct((NP, 128), jnp.float32),
        ),
        mesh=_sc_mesh(),
        scratch_types=[
            pltpu.VMEM((per_w,), jnp.int32),
            pltpu.VMEM((CHUNK,), jnp.int32),
            pltpu.VMEM((CHUNK,), jnp.int32),
            pltpu.VMEM((max(tail, 1),), jnp.int32),
            pltpu.VMEM((CHUNK, 128), jnp.float32),
            pltpu.VMEM((CHUNK, 128), jnp.float32),
            pltpu.VMEM_SHARED((NP, 128), jnp.float32),
            pltpu.SemaphoreType.DMA,
            pltpu.SemaphoreType.DMA,
            pltpu.SemaphoreType.DMA,
            pltpu.SemaphoreType.DMA,
            pltpu.SemaphoreType.DMA,
        ],
    )
    return f(hs, src1d, dst1d)


def _norm_scale_call(deg0, deg1, feat, N):
    R = 400
    grid = (N // R,)

    def body(d0_ref, d1_ref, feat_ref, norm_ref, hs_ref):
        d = d0_ref[...] + d1_ref[...]
        nrm = jnp.where(d > 0.0, lax.rsqrt(d), 0.0)
        norm_ref[...] = nrm
        hs_ref[...] = feat_ref[...] * nrm

    return pl.pallas_call(
        body,
        grid=grid,
        in_specs=[
            pl.BlockSpec((R, 1), lambda i: (i, 0)),
            pl.BlockSpec((R, 1), lambda i: (i, 0)),
            pl.BlockSpec((R, 128), lambda i: (i, 0)),
        ],
        out_specs=[
            pl.BlockSpec((R, 1), lambda i: (i, 0)),
            pl.BlockSpec((R, 128), lambda i: (i, 0)),
        ],
        out_shape=[
            jax.ShapeDtypeStruct((N, 1), jnp.float32),
            jax.ShapeDtypeStruct((N, 128), jnp.float32),
        ],
    )(deg0, deg1, feat)


def _combine_scale_call(a0, a1, norm, N):
    R = 400
    grid = (N // R,)

    def body(a0_ref, a1_ref, norm_ref, hs_ref):
        nrm = norm_ref[...]
        hs_ref[...] = (a0_ref[...] + a1_ref[...]) * (nrm * nrm)

    return pl.pallas_call(
        body,
        grid=grid,
        in_specs=[
            pl.BlockSpec((R, 128), lambda i: (i, 0)),
            pl.BlockSpec((R, 128), lambda i: (i, 0)),
            pl.BlockSpec((R, 1), lambda i: (i, 0)),
        ],
        out_specs=pl.BlockSpec((R, 128), lambda i: (i, 0)),
        out_shape=jax.ShapeDtypeStruct((N, 128), jnp.float32),
    )(a0, a1, norm)


def _final_call(b0, b1, norm, W, b2, N):
    R = 400
    grid = (N // R,)

    def body(b0_ref, b1_ref, norm_ref, w_ref, bias_ref, out_ref):
        h = (b0_ref[...] + b1_ref[...]) * norm_ref[...]
        out_ref[...] = lax.dot_general(
            h, w_ref[...], (((1,), (1,)), ((), ())),
            preferred_element_type=jnp.float32) + bias_ref[...]

    return pl.pallas_call(
        body,
        grid=grid,
        in_specs=[
            pl.BlockSpec((R, 128), lambda i: (i, 0)),
            pl.BlockSpec((R, 128), lambda i: (i, 0)),
            pl.BlockSpec((R, 1), lambda i: (i, 0)),
            pl.BlockSpec((128, 128), lambda i: (0, 0)),
            pl.BlockSpec((1, 128), lambda i: (0, 0)),
        ],
        out_specs=pl.BlockSpec((R, 128), lambda i: (i, 0)),
        out_shape=jax.ShapeDtypeStruct((N, 128), jnp.float32),
    )(b0, b1, norm, W, b2)


def kernel(feat, edge_index, W, b):
    N, D = feat.shape
    E = edge_index.shape[1]
    ei = edge_index.astype(jnp.int32)
    src, dst = ei[0], ei[1]
    zeros1d = jnp.zeros((NP,), jnp.float32)

    n_it = -(-E // (NW * CHUNK * 8)) * 8
    dst2d = _pad_dst(dst, E, N, n_it)

    deg0, deg1 = _deg_call(dst2d, zeros1d)
    norm, hs1 = _norm_scale_call(deg0.reshape(NP, 1), deg1.reshape(NP, 1),
                                 feat, N)
    a0, a1 = _hop_call(hs1, src, dst, E)
    hs2 = _combine_scale_call(a0, a1, norm, N)
    b0, b1 = _hop_call(hs2, src, dst, E)
    return _final_call(b0, b1, norm, W, b.reshape(1, 128), N)


# speedup vs baseline: 11.1379x; 1.1039x over previous
import jax
import jax.numpy as jnp
from jax import lax
from jax.experimental import pallas as pl
from jax.experimental.pallas import tpu as pltpu
from jax.experimental.pallas import tpu_sc as plsc

NC = 2
NS = 16
NW = NC * NS
CHUNK = 128
NP = 10240


def _sc_mesh():
    return plsc.VectorSubcoreMesh(core_axis_name="c", subcore_axis_name="s")


def _pad_dst(dst, E, N, n_it):
    e_pad = NW * n_it * CHUNK
    fill = jnp.arange(e_pad - E, dtype=jnp.int32)
    dstp = jnp.concatenate([dst, N + (fill % (NP - N))])
    return dstp.reshape(e_pad // CHUNK, CHUNK)


def _deg_call(dst2d, zeros1d):
    n_it = dst2d.shape[0] // NW
    rps = NP // NS

    def body(dst_hbm, zeros_hbm, out0_hbm, out1_hbm,
             ones_v, didx2, acc, sem):
        c = lax.axis_index("c")
        s = lax.axis_index("s")
        w = s * NC + c
        sl = pl.ds(s * rps, rps)
        pltpu.sync_copy(zeros_hbm.at[sl], acc.at[sl])
        for i in range(CHUNK // 16):
            ones_v[pl.ds(i * 16, 16)] = jnp.ones((16,), jnp.float32)
        pltpu.sync_copy(dst_hbm.at[pl.ds(w * n_it, n_it)], didx2)
        plsc.subcore_barrier()

        grp = 8

        def it(k, carry):
            for j in range(grp):
                pltpu.async_copy(ones_v, acc.at[didx2.at[k * grp + j]],
                                 sem, add=True)
            for j in range(grp):
                pltpu.make_async_copy(ones_v, acc.at[didx2.at[k * grp + j]],
                                      sem).wait()
            return carry

        lax.fori_loop(0, n_it // grp, it, 0)
        plsc.subcore_barrier()

        @pl.when(c == 0)
        def _():
            pltpu.sync_copy(acc.at[sl], out0_hbm.at[sl])

        @pl.when(c == 1)
        def _():
            pltpu.sync_copy(acc.at[sl], out1_hbm.at[sl])

    f = pl.kernel(
        body,
        out_type=(
            jax.ShapeDtypeStruct((NP,), jnp.float32),
            jax.ShapeDtypeStruct((NP,), jnp.float32),
        ),
        mesh=_sc_mesh(),
        scratch_types=[
            pltpu.VMEM((CHUNK,), jnp.float32),
            pltpu.VMEM((n_it, CHUNK), jnp.int32),
            pltpu.VMEM_SHARED((NP,), jnp.float32),
            pltpu.SemaphoreType.DMA,
        ],
    )
    return f(dst2d, zeros1d)


def _hop_call(hs, src1d, dst1d, E):
    per_w = E // NW
    n_full = per_w // CHUNK
    tail = per_w % CHUNK
    rps = NP // NS

    def body(hs_hbm, src_hbm, dst_hbm, out0_hbm, out1_hbm,
             sidx_buf, didx0, didx1, didx_t, rows0, rows1, acc,
             sem0, sem1, dsem0, dsem1, tsem):
        c = lax.axis_index("c")
        s = lax.axis_index("s")
        w = s * NC + c
        sl = pl.ds(s * rps, rps)
        ebase = w * per_w

        def zrow(r, carry):
            for j in range(8):
                rows0[r, pl.ds(j * 16, 16)] = jnp.zeros((16,), jnp.float32)
            return carry

        lax.fori_loop(0, CHUNK, zrow, 0)
        for k in range(rps // CHUNK):
            pltpu.sync_copy(rows0, acc.at[pl.ds(s * rps + k * CHUNK, CHUNK)])
        pltpu.sync_copy(src_hbm.at[pl.ds(ebase, per_w)], sidx_buf)
        plsc.subcore_barrier()

        rows = (rows0, rows1)
        sems = (sem0, sem1)
        didx = (didx0, didx1)
        dsems = (dsem0, dsem1)

        def sslice(i):
            return sidx_buf.at[pl.ds(i * CHUNK, CHUNK)]

        def dslice(i):
            return dst_hbm.at[pl.ds(ebase + i * CHUNK, CHUNK)]

        pltpu.async_copy(hs_hbm.at[sslice(0)], rows0, sem0)
        pltpu.async_copy(dslice(0), didx0, dsem0)

        def pair(k, carry):
            for b in range(2):
                i = k * 2 + b
                nb = 1 - b
                ip1 = lax.rem(i + 1, n_full)
                pltpu.async_copy(hs_hbm.at[sslice(ip1)], rows[nb],
                                 sems[nb])
                pltpu.async_copy(dslice(ip1), didx[nb], dsems[nb])
                pltpu.make_async_copy(dslice(i), didx[b], dsems[b]).wait()
                pltpu.make_async_copy(hs_hbm.at[sslice(i)], rows[b],
                                      sems[b]).wait()
                pltpu.sync_copy(rows[b], acc.at[didx[b]], add=True)
            return carry

        lax.fori_loop(0, n_full // 2, pair, 0)
        pltpu.make_async_copy(dslice(0), didx[0], dsems[0]).wait()
        pltpu.make_async_copy(hs_hbm.at[sslice(0)], rows[0],
                              sems[0]).wait()
        if tail:
            tbase = ebase + n_full * CHUNK
            pltpu.sync_copy(dst_hbm.at[pl.ds(tbase, tail)], didx_t)
            pltpu.async_copy(
                hs_hbm.at[sidx_buf.at[pl.ds(n_full * CHUNK, tail)]],
                rows1.at[pl.ds(0, tail)], tsem).wait()
            pltpu.sync_copy(rows1.at[pl.ds(0, tail)], acc.at[didx_t],
                            add=True)
        plsc.subcore_barrier()

        @pl.when(c == 0)
        def _():
            pltpu.sync_copy(acc.at[sl], out0_hbm.at[sl])

        @pl.when(c == 1)
        def _():
            pltpu.sync_copy(acc.at[sl], out1_hbm.at[sl])

    f = pl.kernel(
        body,
        out_type=(
            jax.ShapeDtypeStruct((NP, 128), jnp.float32),
            jax.ShapeDtypeStruct((NP, 128), jnp.float32),
        ),
        mesh=_sc_mesh(),
        scratch_types=[
            pltpu.VMEM((per_w,), jnp.int32),
            pltpu.VMEM((CHUNK,), jnp.int32),
            pltpu.VMEM((CHUNK,), jnp.int32),
            pltpu.VMEM((max(tail, 1),), jnp.int32),
            pltpu.VMEM((CHUNK, 128), jnp.float32),
            pltpu.VMEM((CHUNK, 128), jnp.float32),
            pltpu.VMEM_SHARED((NP, 128), jnp.float32),
            pltpu.SemaphoreType.DMA,
            pltpu.SemaphoreType.DMA,
            pltpu.SemaphoreType.DMA,
            pltpu.SemaphoreType.DMA,
            pltpu.SemaphoreType.DMA,
        ],
    )
    return f(hs, src1d, dst1d)


def _norm_scale_call(deg0, deg1, feat, N):
    R = 2000
    grid = (N // R,)

    def body(d0_ref, d1_ref, feat_ref, norm_ref, hs_ref):
        d = d0_ref[...] + d1_ref[...]
        nrm = jnp.where(d > 0.0, lax.rsqrt(d), 0.0)
        norm_ref[...] = nrm
        hs_ref[...] = feat_ref[...] * nrm

    return pl.pallas_call(
        body,
        grid=grid,
        in_specs=[
            pl.BlockSpec((R, 1), lambda i: (i, 0)),
            pl.BlockSpec((R, 1), lambda i: (i, 0)),
            pl.BlockSpec((R, 128), lambda i: (i, 0)),
        ],
        out_specs=[
            pl.BlockSpec((R, 1), lambda i: (i, 0)),
            pl.BlockSpec((R, 128), lambda i: (i, 0)),
        ],
        out_shape=[
            jax.ShapeDtypeStruct((N, 1), jnp.float32),
            jax.ShapeDtypeStruct((N, 128), jnp.float32),
        ],
    )(deg0, deg1, feat)


def _combine_scale_call(a0, a1, norm, N):
    R = 2000
    grid = (N // R,)

    def body(a0_ref, a1_ref, norm_ref, hs_ref):
        nrm = norm_ref[...]
        hs_ref[...] = (a0_ref[...] + a1_ref[...]) * (nrm * nrm)

    return pl.pallas_call(
        body,
        grid=grid,
        in_specs=[
            pl.BlockSpec((R, 128), lambda i: (i, 0)),
            pl.BlockSpec((R, 128), lambda i: (i, 0)),
            pl.BlockSpec((R, 1), lambda i: (i, 0)),
        ],
        out_specs=pl.BlockSpec((R, 128), lambda i: (i, 0)),
        out_shape=jax.ShapeDtypeStruct((N, 128), jnp.float32),
    )(a0, a1, norm)


def _final_call(b0, b1, norm, W, b2, N):
    R = 2000
    grid = (N // R,)

    def body(b0_ref, b1_ref, norm_ref, w_ref, bias_ref, out_ref):
        h = (b0_ref[...] + b1_ref[...]) * norm_ref[...]
        out_ref[...] = lax.dot_general(
            h, w_ref[...], (((1,), (1,)), ((), ())),
            preferred_element_type=jnp.float32) + bias_ref[...]

    return pl.pallas_call(
        body,
        grid=grid,
        in_specs=[
            pl.BlockSpec((R, 128), lambda i: (i, 0)),
            pl.BlockSpec((R, 128), lambda i: (i, 0)),
            pl.BlockSpec((R, 1), lambda i: (i, 0)),
            pl.BlockSpec((128, 128), lambda i: (0, 0)),
            pl.BlockSpec((1, 128), lambda i: (0, 0)),
        ],
        out_specs=pl.BlockSpec((R, 128), lambda i: (i, 0)),
        out_shape=jax.ShapeDtypeStruct((N, 128), jnp.float32),
    )(b0, b1, norm, W, b2)


def kernel(feat, edge_index, W, b):
    N, D = feat.shape
    E = edge_index.shape[1]
    ei = edge_index.astype(jnp.int32)
    src, dst = ei[0], ei[1]
    zeros1d = jnp.zeros((NP,), jnp.float32)

    n_it = -(-E // (NW * CHUNK * 8)) * 8
    dst2d = _pad_dst(dst, E, N, n_it)

    deg0, deg1 = _deg_call(dst2d, zeros1d)
    norm, hs1 = _norm_scale_call(deg0.reshape(NP, 1), deg1.reshape(NP, 1),
                                 feat, N)
    a0, a1 = _hop_call(hs1, src, dst, E)
    hs2 = _combine_scale_call(a0, a1, norm, N)
    b0, b1 = _hop_call(hs2, src, dst, E)
    return _final_call(b0, b1, norm, W, b.reshape(1, 128), N)
